# Initial kernel scaffold; baseline (speedup 1.0000x reference)
import jax
import jax.numpy as jnp
from jax.experimental import pallas as pl


def kernel(membership, table):
    raise NotImplementedError("write your pallas kernel here")


# speedup vs baseline: 8.6425x; 8.6425x over previous
import jax
import jax.numpy as jnp
from jax import lax
from jax.experimental import pallas as pl


def kernel(membership, table):
    B, S, D = membership.shape
    E = table.shape[1]
    N = B * S * D
    PER = 128 // E
    ROWS = N // PER
    BLK = 2048

    m4 = membership.reshape(ROWS, PER).astype(jnp.int32)
    t01 = jnp.tile(table, (1, PER))

    def body(m_ref, t_ref, out_ref):
        m = m_ref[...]
        lane_grp = lax.broadcasted_iota(jnp.int32, (BLK, 128), 1) >> 5
        mi = jnp.zeros((BLK, 128), jnp.int32)
        for j in range(PER):
            mi = jnp.where(lane_grp == j, m[:, j:j + 1], mi)
        t0 = t_ref[0:1, :]
        t1 = t_ref[1:2, :]
        out_ref[...] = jnp.where(mi == 1, t1, t0)

    out2 = pl.pallas_call(
        body,
        grid=(ROWS // BLK,),
        in_specs=[
            pl.BlockSpec((BLK, PER), lambda i: (i, 0)),
            pl.BlockSpec((2, 128), lambda i: (0, 0)),
        ],
        out_specs=pl.BlockSpec((BLK, 128), lambda i: (i, 0)),
        out_shape=jax.ShapeDtypeStruct((ROWS, 128), jnp.float32),
    )(m4, t01)
    return out2.reshape(B, S, D, E)


# speedup vs baseline: 9.1384x; 1.0574x over previous
import jax
import jax.numpy as jnp
from jax import lax
from jax.experimental import pallas as pl


def kernel(membership, table):
    B, S, D = membership.shape
    E = table.shape[1]
    N = B * S * D
    PER = 128 // E
    ROWS = N // PER
    BLK = 2048

    m4 = membership.reshape(ROWS, PER).astype(jnp.int32)
    t01 = jnp.tile(table, (1, PER))

    def body(m_ref, t_ref, out_ref):
        t0 = t_ref[0:1, :]
        d = t_ref[1:2, :] - t0
        lane_grp = lax.broadcasted_iota(jnp.int32, (PER, 128), 1) >> 5
        row_id = lax.broadcasted_iota(jnp.int32, (PER, 128), 0)
        pd = jnp.where(lane_grp == row_id, d, 0.0)
        mf = m_ref[...].astype(jnp.float32)
        out_ref[...] = jnp.dot(
            mf, pd, preferred_element_type=jnp.float32) + t0

    out2 = pl.pallas_call(
        body,
        grid=(ROWS // BLK,),
        in_specs=[
            pl.BlockSpec((BLK, PER), lambda i: (i, 0)),
            pl.BlockSpec((2, 128), lambda i: (0, 0)),
        ],
        out_specs=pl.BlockSpec((BLK, 128), lambda i: (i, 0)),
        out_shape=jax.ShapeDtypeStruct((ROWS, 128), jnp.float32),
    )(m4, t01)
    return out2.reshape(B, S, D, E)


# speedup vs baseline: 12.2647x; 1.3421x over previous
import jax
import jax.numpy as jnp
from jax import lax
from jax.experimental import pallas as pl
from jax.experimental.pallas import tpu as pltpu


def kernel(membership, table):
    B, S, D = membership.shape
    E = table.shape[1]
    N = B * S * D
    WID = 4096
    ROWS = N * E // WID
    RB = 128

    md = membership.reshape(ROWS, 128).astype(jnp.int32)
    t4 = jnp.tile(table, (1, WID // E))

    def body(m_ref, t_ref, out_ref, big_ref):
        @pl.when(pl.program_id(0) == 0)
        def _init():
            ci = lax.broadcasted_iota(jnp.int32, (128, WID), 0)
            ki = lax.broadcasted_iota(jnp.int32, (128, WID), 1) >> 5
            d = t_ref[1:2, :] - t_ref[0:1, :]
            big_ref[...] = jnp.where(ci == ki, d, 0.0)

        mf = m_ref[...].astype(jnp.float32)
        out_ref[...] = jnp.dot(
            mf, big_ref[...], preferred_element_type=jnp.float32
        ) + t_ref[0:1, :]

    out2 = pl.pallas_call(
        body,
        grid=(ROWS // RB,),
        in_specs=[
            pl.BlockSpec((RB, 128), lambda i: (i, 0)),
            pl.BlockSpec((2, WID), lambda i: (0, 0)),
        ],
        out_specs=pl.BlockSpec((RB, WID), lambda i: (i, 0)),
        out_shape=jax.ShapeDtypeStruct((ROWS, WID), jnp.float32),
        scratch_shapes=[pltpu.VMEM((128, WID), jnp.float32)],
    )(md, t4)
    return out2.reshape(B, S, D, E)
